# baseline (speedup 1.0000x reference)
import functools

import jax
import jax.numpy as jnp
from jax import lax
from jax.experimental import pallas as pl
from jax.experimental.pallas import tpu as pltpu
from jax.experimental.pallas import tpu_sc as plsc

_CHUNK = 128
_K = 64


@functools.cache
def _make_gather(v, d, bsz, hist):
    info = plsc.get_sparse_core_info()
    nc, ns = info.num_cores, info.num_subcores
    nw = nc * ns
    chunk = _CHUNK
    lanes_per_worker = bsz // nw
    assert lanes_per_worker == chunk
    mesh = plsc.VectorSubcoreMesh(core_axis_name="c", subcore_axis_name="s")

    @functools.partial(
        pl.kernel,
        mesh=mesh,
        compiler_params=pltpu.CompilerParams(use_tc_tiling_on_sc=False),
        out_type=jax.ShapeDtypeStruct((2 * bsz * hist, d), jnp.bfloat16),
        scratch_types=[
            pltpu.VMEM((hist, chunk), jnp.int32),
            pltpu.VMEM((chunk, d), jnp.bfloat16),
            pltpu.VMEM((chunk, d), jnp.bfloat16),
            pltpu.VMEM((1, chunk), jnp.int32),
            pltpu.VMEM((1, chunk), jnp.int32),
            pltpu.SemaphoreType.DMA,
            pltpu.SemaphoreType.DMA,
            pltpu.SemaphoreType.DMA,
            pltpu.SemaphoreType.DMA,
        ],
    )
    def gather_kernel(
        table_hbm, idst_hbm, out_hbm,
        idx_v, rows_a, rows_b, sidx_a, sidx_b, gsem_a, gsem_b, ssem_a, ssem_b,
    ):
        wid = lax.axis_index("s") * nc + lax.axis_index("c")
        col0 = wid * chunk
        pltpu.sync_copy(idst_hbm.at[:, pl.ds(col0, chunk)], idx_v)

        def fill_sidx(sidx, j):
            base = 2 * (j * bsz + col0)
            for k in range(chunk // 16):
                sidx[0, pl.ds(k * 16, 16)] = (
                    lax.iota(jnp.int32, 16) * 2 + (base + 32 * k)
                )

        def body(j2, carry):
            c0 = 2 * j2
            c1 = c0 + 1
            pltpu.async_copy(table_hbm.at[idx_v.at[c0]], rows_a, gsem_a)
            pltpu.async_copy(table_hbm.at[idx_v.at[c1]], rows_b, gsem_b)
            fill_sidx(sidx_a, c0)
            fill_sidx(sidx_b, c1)
            pltpu.make_async_copy(table_hbm.at[idx_v.at[c0]], rows_a, gsem_a).wait()
            sca = pltpu.async_copy(rows_a, out_hbm.at[sidx_a.at[0]], ssem_a)
            pltpu.make_async_copy(table_hbm.at[idx_v.at[c1]], rows_b, gsem_b).wait()
            scb = pltpu.async_copy(rows_b, out_hbm.at[sidx_b.at[0]], ssem_b)
            sca.wait()
            scb.wait()
            return carry

        lax.fori_loop(0, hist // 2, body, 0)

    return gather_kernel


def _assemble_body(g_ref, idst_ref, pids_ref, featp_ref, wg_ref, out_ref):
    corr_tbl = lax.dot_general(
        featp_ref[...], wg_ref[...],
        (((1,), (1,)), ((), ())),
        preferred_element_type=jnp.float32,
        precision=lax.Precision.HIGHEST,
    )
    ids_row = idst_ref[pl.ds(pl.program_id(0), 1), :]
    pids_t = pids_ref[...].T
    onehot = (pids_t == ids_row).astype(jnp.float32)
    corr_t = lax.dot_general(
        corr_tbl, onehot,
        (((0,), (0,)), ((), ())),
        preferred_element_type=jnp.float32,
        precision=lax.Precision.HIGHEST,
    )
    d = featp_ref.shape[1]
    g_t = g_ref[...][:, :d].astype(jnp.float32).T
    out_ref[0, :, :] = g_t + corr_t


def _assemble(g_view, idst, pids, featp, wg, bsz, hist, d):
    return pl.pallas_call(
        _assemble_body,
        grid=(hist,),
        in_specs=[
            pl.BlockSpec((bsz, 2 * d), lambda h: (h, 0)),
            pl.BlockSpec((hist, bsz), lambda h: (0, 0)),
            pl.BlockSpec((1, _K), lambda h: (0, 0)),
            pl.BlockSpec((_K, d), lambda h: (0, 0)),
            pl.BlockSpec((d, d), lambda h: (0, 0)),
        ],
        out_specs=pl.BlockSpec((1, d, bsz), lambda h: (h, 0, 0)),
        out_shape=jax.ShapeDtypeStruct((hist, d, bsz), jnp.float32),
    )(g_view, idst, pids, featp, wg)


def kernel(ids, id_embed, feat_tbl, W, gamma, prod_mask):
    v, d = id_embed.shape
    bsz, hist = ids.shape

    pidx = jnp.nonzero(prod_mask, size=_K, fill_value=0)[0].astype(jnp.int32)
    count = jnp.sum(prod_mask.astype(jnp.int32))
    pids = jnp.where(jnp.arange(_K, dtype=jnp.int32) < count, pidx, -1)
    pids = pids.reshape(1, _K)
    featp = jnp.take(feat_tbl, pidx, axis=0)
    wg = W * gamma.astype(jnp.float32)

    idst = ids.astype(jnp.int32).T

    gather_fn = _make_gather(v, d, bsz, hist)
    scat = gather_fn(id_embed.astype(jnp.bfloat16), idst)
    g_view = jnp.reshape(scat, (bsz * hist, 2 * d))

    out_t = _assemble(g_view, idst, pids, featp, wg, bsz, hist, d)
    return jnp.transpose(out_t, (2, 0, 1))


# speedup vs baseline: 2.2558x; 2.2558x over previous
import functools

import jax
import jax.numpy as jnp
from jax import lax
from jax.experimental import pallas as pl
from jax.experimental.pallas import tpu as pltpu
from jax.experimental.pallas import tpu_sc as plsc

_CHUNK = 128
_K = 64


@functools.cache
def _make_gather(v, d, bsz, hist):
    info = plsc.get_sparse_core_info()
    nc, ns = info.num_cores, info.num_subcores
    nw = nc * ns
    chunk = _CHUNK
    lanes_per_worker = bsz // nw
    assert lanes_per_worker == chunk
    mesh = plsc.VectorSubcoreMesh(core_axis_name="c", subcore_axis_name="s")

    @functools.partial(
        pl.kernel,
        mesh=mesh,
        compiler_params=pltpu.CompilerParams(use_tc_tiling_on_sc=False),
        out_type=jax.ShapeDtypeStruct((bsz * hist, d), jnp.float32),
        scratch_types=[
            pltpu.VMEM((hist, chunk), jnp.int32),
            pltpu.VMEM((chunk, d), jnp.float32),
            pltpu.VMEM((chunk, d), jnp.float32),
            pltpu.VMEM((1, chunk), jnp.int32),
            pltpu.VMEM((1, chunk), jnp.int32),
            pltpu.SemaphoreType.DMA,
            pltpu.SemaphoreType.DMA,
            pltpu.SemaphoreType.DMA,
            pltpu.SemaphoreType.DMA,
        ],
    )
    def gather_kernel(
        table_hbm, idst_hbm, out_hbm,
        idx_v, rows_a, rows_b, sidx_a, sidx_b, gsem_a, gsem_b, ssem_a, ssem_b,
    ):
        wid = lax.axis_index("s") * nc + lax.axis_index("c")
        col0 = wid * chunk
        pltpu.sync_copy(idst_hbm.at[:, pl.ds(col0, chunk)], idx_v)

        def fill_sidx(sidx, j2, parity):
            base = 2 * (j2 * bsz + col0) + parity
            for k in range(chunk // 16):
                sidx[0, pl.ds(k * 16, 16)] = (
                    lax.iota(jnp.int32, 16) * 2 + (base + 32 * k)
                )

        def body(j2, carry):
            c0 = 2 * j2
            c1 = c0 + 1
            pltpu.async_copy(table_hbm.at[idx_v.at[c0]], rows_a, gsem_a)
            pltpu.async_copy(table_hbm.at[idx_v.at[c1]], rows_b, gsem_b)
            fill_sidx(sidx_a, j2, 0)
            fill_sidx(sidx_b, j2, 1)
            pltpu.make_async_copy(table_hbm.at[idx_v.at[c0]], rows_a, gsem_a).wait()
            sca = pltpu.async_copy(rows_a, out_hbm.at[sidx_a.at[0]], ssem_a)
            pltpu.make_async_copy(table_hbm.at[idx_v.at[c1]], rows_b, gsem_b).wait()
            scb = pltpu.async_copy(rows_b, out_hbm.at[sidx_b.at[0]], ssem_b)
            sca.wait()
            scb.wait()
            return carry

        lax.fori_loop(0, hist // 2, body, 0)

    return gather_kernel


def _assemble_body(g_ref, idst_ref, pids_ref, featp_ref, wg_ref, out_ref):
    corr_tbl = lax.dot_general(
        featp_ref[...], wg_ref[...],
        (((1,), (1,)), ((), ())),
        preferred_element_type=jnp.float32,
        precision=lax.Precision.HIGHEST,
    )
    pids_t = pids_ref[...].T
    d = featp_ref.shape[1]
    h2 = pl.program_id(0)
    g = g_ref[...]
    for parity in range(2):
        ids_row = idst_ref[pl.ds(2 * h2 + parity, 1), :]
        onehot = (pids_t == ids_row).astype(jnp.float32)
        corr_t = lax.dot_general(
            corr_tbl, onehot,
            (((0,), (0,)), ((), ())),
            preferred_element_type=jnp.float32,
            precision=lax.Precision.HIGHEST,
        )
        g_t = g[:, parity * d:(parity + 1) * d].T
        out_ref[parity, :, :] = g_t + corr_t


def _assemble(g_view, idst, pids, featp, wg, bsz, hist, d):
    return pl.pallas_call(
        _assemble_body,
        grid=(hist // 2,),
        in_specs=[
            pl.BlockSpec((bsz, 2 * d), lambda h: (h, 0)),
            pl.BlockSpec((hist, bsz), lambda h: (0, 0)),
            pl.BlockSpec((1, _K), lambda h: (0, 0)),
            pl.BlockSpec((_K, d), lambda h: (0, 0)),
            pl.BlockSpec((d, d), lambda h: (0, 0)),
        ],
        out_specs=pl.BlockSpec((2, d, bsz), lambda h: (h, 0, 0)),
        out_shape=jax.ShapeDtypeStruct((hist, d, bsz), jnp.float32),
    )(g_view, idst, pids, featp, wg)


def kernel(ids, id_embed, feat_tbl, W, gamma, prod_mask):
    v, d = id_embed.shape
    bsz, hist = ids.shape

    pidx = jnp.nonzero(prod_mask, size=_K, fill_value=0)[0].astype(jnp.int32)
    count = jnp.sum(prod_mask.astype(jnp.int32))
    pids = jnp.where(jnp.arange(_K, dtype=jnp.int32) < count, pidx, -1)
    pids = pids.reshape(1, _K)
    featp = jnp.take(feat_tbl, pidx, axis=0)
    wg = W * gamma.astype(jnp.float32)

    idst = ids.astype(jnp.int32).T

    gather_fn = _make_gather(v, d, bsz, hist)
    scat = gather_fn(id_embed, idst)
    g_view = jnp.reshape(scat, (bsz * hist // 2, 2 * d))

    out_t = _assemble(g_view, idst, pids, featp, wg, bsz, hist, d)
    return jnp.transpose(out_t, (2, 0, 1))
